# Initial kernel scaffold; baseline (speedup 1.0000x reference)
import jax
import jax.numpy as jnp
from jax.experimental import pallas as pl


def kernel(node_scalars, rigids_rot, rigids_trans, edge_features, edge_index, node_vectors, W_pts, W_comb, W_mlp, b_mlp):
    raise NotImplementedError("write your pallas kernel here")


# speedup vs baseline: 5.0094x; 5.0094x over previous
import functools

import jax
import jax.numpy as jnp
from jax import lax
from jax.experimental import pallas as pl
from jax.experimental.pallas import tpu as pltpu
from jax.experimental.pallas import tpu_sc as plsc

N_NODES = 10000
N_EDGES = 320000
C_S = 128
C_V = 8
C_Z = 16
NS_PTS = 8
D_POINTS = 10.0

_NC = 2
_NS = 16
_NW = _NC * _NS
_EPW = N_EDGES // _NW
_CH = 80
_NCHUNK = _EPW // _CH
_N_PAD = 10240
_ROWS_PER_TILE = _N_PAD // _NS


def _node_kernel(ns_ref, rot_ref, tr_ref, nvt_ref, wpts_ref, wcomb_ref,
                 w1_ref, w2_ref, w46_ref, w56_ref, b_ref,
                 vcat_ref, a_ref, b_out_ref):
    ns = ns_ref[...]
    rot = rot_ref[...]
    tr = tr_ref[...]
    nvt = nvt_ref[...]
    wpts = wpts_ref[...]
    wcomb = wcomb_ref[...]
    wc_top = wcomb[:C_V, :]
    wc_bot = wcomb[C_V:, :]
    p = jnp.dot(ns, wpts, preferred_element_type=jnp.float32)
    a_acc = jnp.dot(ns, w1_ref[...], preferred_element_type=jnp.float32)
    a_acc = a_acc + b_ref[...]
    b_acc = jnp.dot(ns, w2_ref[...], preferred_element_type=jnp.float32)
    for i in range(3):
        r_i = (rot[:, 3 * i + 0:3 * i + 1] * p[:, 0:NS_PTS]
               + rot[:, 3 * i + 1:3 * i + 2] * p[:, NS_PTS:2 * NS_PTS]
               + rot[:, 3 * i + 2:3 * i + 3] * p[:, 2 * NS_PTS:3 * NS_PTS])
        nv_i = nvt[:, C_V * i:C_V * (i + 1)]
        out_v_i = (jnp.dot(nv_i, wc_top, preferred_element_type=jnp.float32)
                   + jnp.dot(r_i, wc_bot, preferred_element_type=jnp.float32)
                   + tr[:, i:i + 1] * (1.0 / D_POINTS))
        vcat_ref[:, C_V * i:C_V * (i + 1)] = out_v_i
        a_acc = a_acc + jnp.dot(out_v_i, w46_ref[C_V * i:C_V * (i + 1), :],
                                preferred_element_type=jnp.float32)
        b_acc = b_acc + jnp.dot(out_v_i, w56_ref[C_V * i:C_V * (i + 1), :],
                                preferred_element_type=jnp.float32)
    a_ref[...] = a_acc
    b_out_ref[...] = b_acc


def _node_precompute(ns, rot9, trans, nvt, W_pts, W_comb, W1, W2, W46, W56, b):
    rows = 1000
    grid = N_NODES // rows
    full = lambda shape: pl.BlockSpec(shape, lambda i: (0, 0))
    blk = lambda w: pl.BlockSpec((rows, w), lambda i: (i, 0))
    return pl.pallas_call(
        _node_kernel,
        grid=(grid,),
        in_specs=[
            blk(C_S), blk(9), blk(3), blk(3 * C_V),
            full((C_S, 3 * NS_PTS)), full((C_V + NS_PTS, C_V)),
            full((C_S, C_S)), full((C_S, C_S)),
            full((3 * C_V, C_S)), full((3 * C_V, C_S)), full((1, C_S)),
        ],
        out_specs=[blk(3 * C_V), blk(C_S), blk(C_S)],
        out_shape=[
            jax.ShapeDtypeStruct((N_NODES, 3 * C_V), jnp.float32),
            jax.ShapeDtypeStruct((N_NODES, C_S), jnp.float32),
            jax.ShapeDtypeStruct((N_NODES, C_S), jnp.float32),
        ],
    )(ns, rot9, trans, nvt, W_pts, W_comb, W1, W2, W46, W56, b)


def _edgec_kernel(ef_ref, w3_ref, c_ref):
    c_ref[...] = jnp.dot(ef_ref[...], w3_ref[...],
                         preferred_element_type=jnp.float32)


def _edge_c(ef, W3):
    rows = 4000
    grid = N_EDGES // rows
    return pl.pallas_call(
        _edgec_kernel,
        grid=(grid,),
        in_specs=[
            pl.BlockSpec((rows, C_Z), lambda i: (i, 0)),
            pl.BlockSpec((C_Z, C_S), lambda i: (0, 0)),
        ],
        out_specs=pl.BlockSpec((rows, C_S), lambda i: (i, 0)),
        out_shape=jax.ShapeDtypeStruct((N_EDGES, C_S), jnp.float32),
    )(ef, W3)


def _softplus_vec(x):
    t = jnp.exp(-jnp.abs(x))
    p = t * (0.9954273 + t * (-0.46407258 + t * (0.21641044
                                                 + t * (-0.05486285))))
    return jnp.maximum(x, 0.0) + p


def _sc_edge_body(a_hbm, b_hbm, c_hbm, src_hbm, dst_hbm, z_hbm, out_hbm,
                  isrc, idst, arows, brows, crows, acc, sem_a, sem_b, sem_c):
    cid = lax.axis_index("c")
    sid = lax.axis_index("s")
    wid = cid * _NS + sid

    r0 = sid * _ROWS_PER_TILE
    pltpu.sync_copy(z_hbm.at[pl.ds(r0, _ROWS_PER_TILE)],
                    acc.at[pl.ds(r0, _ROWS_PER_TILE)])
    plsc.subcore_barrier()

    def chunk_body(j, carry):
        base = wid * _EPW + j * _CH
        pltpu.sync_copy(src_hbm.at[pl.ds(base, _CH)], isrc)
        pltpu.sync_copy(dst_hbm.at[pl.ds(base, _CH)], idst)
        ca = pltpu.async_copy(a_hbm.at[isrc], arows, sem_a)
        cb = pltpu.async_copy(b_hbm.at[idst], brows, sem_b)
        cc = pltpu.async_copy(c_hbm.at[pl.ds(base, _CH)], crows, sem_c)
        ca.wait()
        cb.wait()
        cc.wait()

        def row_body(r, rcarry):
            for v in range(C_S // 16):
                sl = pl.ds(v * 16, 16)
                x = arows[r, sl] + brows[r, sl] + crows[r, sl]
                arows[r, sl] = _softplus_vec(x)
            return rcarry

        lax.fori_loop(0, _CH, row_body, 0)
        pltpu.sync_copy(arows, acc.at[idst], add=True)
        return carry

    lax.fori_loop(0, _NCHUNK, chunk_body, 0)
    plsc.subcore_barrier()
    pltpu.sync_copy(acc.at[pl.ds(r0, _ROWS_PER_TILE)],
                    out_hbm.at[cid, pl.ds(r0, _ROWS_PER_TILE)])


_sc_edge = functools.partial(
    pl.kernel,
    out_type=jax.ShapeDtypeStruct((_NC, _N_PAD, C_S), jnp.float32),
    mesh=plsc.VectorSubcoreMesh(core_axis_name="c", subcore_axis_name="s"),
    scratch_types=[
        pltpu.VMEM((_CH,), jnp.int32),
        pltpu.VMEM((_CH,), jnp.int32),
        pltpu.VMEM((_CH, C_S), jnp.float32),
        pltpu.VMEM((_CH, C_S), jnp.float32),
        pltpu.VMEM((_CH, C_S), jnp.float32),
        pltpu.VMEM_SHARED((_N_PAD, C_S), jnp.float32),
        pltpu.SemaphoreType.DMA,
        pltpu.SemaphoreType.DMA,
        pltpu.SemaphoreType.DMA,
    ],
)(_sc_edge_body)


def _sum2_kernel(p_ref, o_ref):
    o_ref[...] = p_ref[0] + p_ref[1]


def _sum_partials(partials):
    rows = 1000
    grid = N_NODES // rows
    return pl.pallas_call(
        _sum2_kernel,
        grid=(grid,),
        in_specs=[pl.BlockSpec((_NC, rows, C_S), lambda i: (0, i, 0))],
        out_specs=pl.BlockSpec((rows, C_S), lambda i: (i, 0)),
        out_shape=jax.ShapeDtypeStruct((N_NODES, C_S), jnp.float32),
    )(partials)


def kernel(node_scalars, rigids_rot, rigids_trans, edge_features, edge_index,
           node_vectors, W_pts, W_comb, W_mlp, b_mlp):
    n = node_scalars.shape[0]
    w1 = W_mlp[:C_S]
    w2 = W_mlp[C_S:2 * C_S]
    w3 = W_mlp[2 * C_S:2 * C_S + C_Z]
    w4 = W_mlp[2 * C_S + C_Z:2 * C_S + C_Z + 3 * C_V]
    w5 = W_mlp[2 * C_S + C_Z + 3 * C_V:2 * C_S + C_Z + 6 * C_V]
    w6 = W_mlp[2 * C_S + C_Z + 6 * C_V:]
    to_cmajor = lambda w: (w.reshape(C_V, 3, C_S).transpose(1, 0, 2)
                           .reshape(3 * C_V, C_S))
    w46 = to_cmajor(w4 + w6)
    w56 = to_cmajor(w5 - w6)
    rot9 = rigids_rot.reshape(n, 9)
    nvt = node_vectors.transpose(0, 2, 1).reshape(n, 3 * C_V)

    vcat, a_tab, b_tab = _node_precompute(
        node_scalars, rot9, rigids_trans, nvt, W_pts, W_comb,
        w1, w2, w46, w56, b_mlp.reshape(1, C_S))
    c_tab = _edge_c(edge_features, w3)

    ei = edge_index.astype(jnp.int32)
    src = ei[0]
    dst = ei[1]
    zeros = jnp.zeros((_N_PAD, C_S), jnp.float32)
    partials = _sc_edge(a_tab, b_tab, c_tab, src, dst, zeros)
    out_s_s = _sum_partials(partials)
    out_s_v = vcat.reshape(n, 3, C_V).transpose(0, 2, 1)
    return (out_s_s, out_s_v)
